# baseline (speedup 1.0000x reference)
import functools

import jax
import jax.numpy as jnp
from jax.experimental import pallas as pl
from jax.experimental.pallas import tpu as pltpu

N = 512
D = 128
T = 4
GH = 64
NE = 64
SE = 128
NC = 10

_F32 = jnp.float32


def _mm(a, b, dims):
    return jax.lax.dot_general(
        a, b, dimension_numbers=(dims, ((), ())),
        preferred_element_type=_F32)


def _mmb(a, b, dims):
    return jax.lax.dot_general(
        a.astype(jnp.bfloat16), b.astype(jnp.bfloat16),
        dimension_numbers=(dims, ((), ())),
        preferred_element_type=_F32)


def _sig(x):
    return 0.5 * jnp.tanh(0.5 * x) + 0.5


def _body(bg1_ref, W_hh1_ref, W1_ref, b1_ref, W2_ref,
          b2_ref, Wsw_ref, bsw_ref, W_ih2_ref, W_hh2_ref, bg2_ref,
          Wc_ref, bc_ref, x_hbm_ref, W_ih1_hbm_ref, adj_hbm_ref, out_ref,
          gin0, gin1, gin2, gin3, adj_ref, adj_sem, x_ref, x_sem,
          W_ih1_ref, w_sem):
    gin_scr = (gin0, gin1, gin2, gin3)
    w_copy = pltpu.make_async_copy(W_ih1_hbm_ref, W_ih1_ref, w_sem)
    w_copy.start()
    x_copies = [
        pltpu.make_async_copy(x_hbm_ref.at[t], x_ref.at[t], x_sem.at[t])
        for t in range(T)]
    for cp in x_copies:
        cp.start()
    adj_copies = [
        pltpu.make_async_copy(adj_hbm_ref.at[t], adj_ref.at[t],
                              adj_sem.at[t])
        for t in range(T)]
    for cp in adj_copies:
        cp.start()
    W1 = W1_ref[...]
    W2 = W2_ref[...]
    W_hh1 = W_hh1_ref[...]

    CH = N // 4
    rl = jax.lax.broadcasted_iota(jnp.int32, (CH, N), 0)
    cl = jax.lax.broadcasted_iota(jnp.int32, (CH, N), 1)
    eye_c = [rl + c * CH == cl for c in range(4)]
    ones_col = jnp.ones((CH, 1), _F32)

    cidx = jax.lax.broadcasted_iota(jnp.int32, (1, 4 * SE), 1)
    gscale = jnp.where(
        jnp.logical_and(cidx >= 2 * SE, cidx < 3 * SE), 1.0, 0.5)
    w_copy.wait()
    for t in range(T):
        x_copies[t].wait()
        gin_scr[t][...] = (_mmb(x_ref[t], W_ih1_ref[...], ((0,), (1,)))
                           + bg1_ref[...]) * gscale

    gvs = {}

    def gcn_gen(t):
        adj_copies[t].wait()
        yield
        Awbs = []
        deg = jnp.zeros((N, 1), _F32)
        for c in range(4):
            a = adj_ref[t, c * CH:(c + 1) * CH, :]
            Aw_c = jnp.where(a > 0, a, 0.0)
            diag = jnp.sum(jnp.where(eye_c[c], Aw_c, 0.0), axis=1,
                           keepdims=True)
            Aw_c = Aw_c + jnp.where(
                eye_c[c], jnp.where(diag > 0, 0.0, 1.0), 0.0)
            deg = deg + _mm(Aw_c, ones_col, ((0,), (0,)))
            Awbs.append(Aw_c.astype(jnp.bfloat16))
            yield
        dinv = jnp.where(deg > 0, jax.lax.rsqrt(deg), 0.0)
        yield
        h = _mmb(x_ref[t], W1, ((1,), (0,)))
        yield
        for W_next, b in ((W2, b1_ref), (None, b2_ref)):
            z = dinv * h
            acc = _mmb(Awbs[0], z[:CH], ((0,), (0,)))
            yield
            for c in range(1, 4):
                acc = acc + _mmb(Awbs[c], z[c * CH:(c + 1) * CH],
                                 ((0,), (0,)))
                yield
            h = jax.nn.relu(dinv * acc + b[...])
            yield
            if W_next is not None:
                h = _mmb(h, W_next, ((1,), (0,)))
                yield
        gvs[t] = jnp.mean(h, axis=0, keepdims=True)
        yield

    gidx = jax.lax.broadcasted_iota(jnp.int32, (4 * SE, 1), 0)
    in_gg = jnp.logical_and(gidx >= 2 * SE, gidx < 3 * SE)
    W_hh1b = (W_hh1 * jnp.where(in_gg, 1.0, 0.5)).astype(jnp.bfloat16)

    def substep(gin, H, C):
        g = gin + _mmb(H, W_hh1b, ((1,), (1,)))
        th = jnp.tanh(g)
        i = 0.5 * th[:, :SE] + 0.5
        f = 0.5 * th[:, SE:2 * SE] + 0.5
        gg = th[:, 2 * SE:3 * SE]
        o = 0.5 * th[:, 3 * SE:] + 0.5
        C = f * C + i * gg
        return o * jnp.tanh(C), C

    H = jnp.zeros((T, SE), _F32)
    C = jnp.zeros((T, SE), _F32)
    gens = [gcn_gen(t) for t in range(T)]
    starts = (16, 44, 72, 100)
    for s in range(D):
        gin = jnp.concatenate([g[s:s + 1, :] for g in gin_scr], axis=0)
        H, C = substep(gin, H, C)
        for t in range(T):
            if s >= starts[t]:
                next(gens[t], None)
    for g in gens:
        for _ in g:
            pass
    H1 = H
    gvs = [gvs[t] for t in range(T)]

    sg = jnp.concatenate([H1, jnp.concatenate(gvs, axis=0)], axis=1)
    sv = jax.nn.relu(_mm(sg, Wsw_ref[...], ((1,), (0,))) + bsw_ref[...])

    W_hh2 = W_hh2_ref[...]
    gin2 = _mm(sv, W_ih2_ref[...], ((1,), (1,))) + bg2_ref[...]
    h = jnp.zeros((1, SE), _F32)
    c = jnp.zeros((1, SE), _F32)
    for s in range(T):
        g = gin2[s:s + 1] + _mm(h, W_hh2, ((1,), (1,)))
        i = _sig(g[:, :SE])
        f = _sig(g[:, SE:2 * SE])
        gg = jnp.tanh(g[:, 2 * SE:3 * SE])
        o = _sig(g[:, 3 * SE:])
        c = f * c + i * gg
        h = o * jnp.tanh(c)

    logits = _mm(h, Wc_ref[...], ((1,), (0,))) + bc_ref[...]
    m = jnp.max(logits, axis=1, keepdims=True)
    z = logits - m
    out_ref[...] = z - jnp.log(jnp.sum(jnp.exp(z), axis=1, keepdims=True))


def _forward(adj_mat_array, node_att_array, W1, b1, W2, b2, W_ih1, W_hh1,
             b_ih1, b_hh1, Wsw, bsw, W_ih2, W_hh2, b_ih2, b_hh2, Wc, bc,
             interpret=False):
    bg1 = (b_ih1 + b_hh1).reshape(1, -1)
    bg2 = (b_ih2 + b_hh2).reshape(1, -1)
    vmem = pl.BlockSpec(memory_space=pltpu.MemorySpace.VMEM)
    hbm = pl.BlockSpec(memory_space=pltpu.MemorySpace.HBM)
    return pl.pallas_call(
        _body,
        out_shape=jax.ShapeDtypeStruct((1, NC), _F32),
        in_specs=[vmem] * 13 + [hbm] * 3,
        scratch_shapes=[pltpu.VMEM((D, 4 * SE), _F32) for _ in range(T)]
        + [pltpu.VMEM((T, N, N), _F32), pltpu.SemaphoreType.DMA((T,)),
           pltpu.VMEM((T, N, D), _F32), pltpu.SemaphoreType.DMA((T,)),
           pltpu.VMEM((4 * SE, N), _F32), pltpu.SemaphoreType.DMA],
        interpret=interpret,
    )(bg1, W_hh1, W1, b1.reshape(1, -1), W2,
      b2.reshape(1, -1), Wsw, bsw.reshape(1, -1), W_ih2, W_hh2, bg2,
      Wc, bc.reshape(1, -1), node_att_array, W_ih1, adj_mat_array)


def kernel(adj_mat_array, node_att_array, W1, b1, W2, b2, W_ih1, W_hh1,
           b_ih1, b_hh1, Wsw, bsw, W_ih2, W_hh2, b_ih2, b_hh2, Wc, bc):
    return _forward(adj_mat_array, node_att_array, W1, b1, W2, b2, W_ih1,
                    W_hh1, b_ih1, b_hh1, Wsw, bsw, W_ih2, W_hh2, b_ih2,
                    b_hh2, Wc, bc)


# speedup vs baseline: 1.0614x; 1.0614x over previous
import functools

import jax
import jax.numpy as jnp
from jax.experimental import pallas as pl
from jax.experimental.pallas import tpu as pltpu

N = 512
D = 128
T = 4
GH = 64
NE = 64
SE = 128
NC = 10

_F32 = jnp.float32


def _mm(a, b, dims):
    return jax.lax.dot_general(
        a, b, dimension_numbers=(dims, ((), ())),
        preferred_element_type=_F32)


def _mmb(a, b, dims):
    return jax.lax.dot_general(
        a.astype(jnp.bfloat16), b.astype(jnp.bfloat16),
        dimension_numbers=(dims, ((), ())),
        preferred_element_type=_F32)


def _sig(x):
    return 0.5 * jnp.tanh(0.5 * x) + 0.5


def _body(x_ref, W_ih1_ref, bg1_ref, W_hh1_ref, W1_ref, b1_ref, W2_ref,
          b2_ref, Wsw_ref, bsw_ref, W_ih2_ref, W_hh2_ref, bg2_ref,
          Wc_ref, bc_ref, adj_hbm_ref, out_ref, gin0, gin1, gin2, gin3,
          adj_ref, adj_sem):
    gin_scr = (gin0, gin1, gin2, gin3)
    adj_copies = [
        pltpu.make_async_copy(adj_hbm_ref.at[t], adj_ref.at[t],
                              adj_sem.at[t])
        for t in range(T)]
    for cp in adj_copies:
        cp.start()
    W1 = W1_ref[...]
    W2 = W2_ref[...]
    W_hh1 = W_hh1_ref[...]

    CH = N // 4
    rl = jax.lax.broadcasted_iota(jnp.int32, (CH, N), 0)
    cl = jax.lax.broadcasted_iota(jnp.int32, (CH, N), 1)
    eye_c = [rl + c * CH == cl for c in range(4)]
    ones_col = jnp.ones((CH, 1), _F32)

    cidx = jax.lax.broadcasted_iota(jnp.int32, (1, 4 * SE), 1)
    gscale = jnp.where(
        jnp.logical_and(cidx >= 2 * SE, cidx < 3 * SE), 1.0, 0.5)
    GB = 32

    def gin_block(t, b):
        sl = slice(b * GB, (b + 1) * GB)
        gin_scr[t][sl, :] = (
            _mmb(x_ref[t][:, sl], W_ih1_ref[...], ((0,), (1,)))
            + bg1_ref[...]) * gscale

    for t in range(T):
        gin_block(t, 0)

    gvs = {}

    def gcn_gen(t):
        adj_copies[t].wait()
        yield
        Awbs = []
        deg = jnp.zeros((N, 1), _F32)
        for c in range(4):
            a = adj_ref[t, c * CH:(c + 1) * CH, :]
            Aw_c = jnp.where(a > 0, a, 0.0)
            diag = jnp.sum(jnp.where(eye_c[c], Aw_c, 0.0), axis=1,
                           keepdims=True)
            Aw_c = Aw_c + jnp.where(
                eye_c[c], jnp.where(diag > 0, 0.0, 1.0), 0.0)
            deg = deg + _mm(Aw_c, ones_col, ((0,), (0,)))
            Awbs.append(Aw_c.astype(jnp.bfloat16))
            yield
        dinv = jnp.where(deg > 0, jax.lax.rsqrt(deg), 0.0)
        yield
        h = _mmb(x_ref[t], W1, ((1,), (0,)))
        yield
        for W_next, b in ((W2, b1_ref), (None, b2_ref)):
            z = dinv * h
            acc = _mmb(Awbs[0], z[:CH], ((0,), (0,)))
            yield
            for c in range(1, 4):
                acc = acc + _mmb(Awbs[c], z[c * CH:(c + 1) * CH],
                                 ((0,), (0,)))
                yield
            h = jax.nn.relu(dinv * acc + b[...])
            yield
            if W_next is not None:
                h = _mmb(h, W_next, ((1,), (0,)))
                yield
        gvs[t] = jnp.mean(h, axis=0, keepdims=True)
        yield

    gidx = jax.lax.broadcasted_iota(jnp.int32, (4 * SE, 1), 0)
    in_gg = jnp.logical_and(gidx >= 2 * SE, gidx < 3 * SE)
    W_hh1b = (W_hh1 * jnp.where(in_gg, 1.0, 0.5)).astype(jnp.bfloat16)

    def substep(gin, H, C):
        g = gin + _mmb(H, W_hh1b, ((1,), (1,)))
        th = jnp.tanh(g)
        i = 0.5 * th[:, :SE] + 0.5
        f = 0.5 * th[:, SE:2 * SE] + 0.5
        gg = th[:, 2 * SE:3 * SE]
        o = 0.5 * th[:, 3 * SE:] + 0.5
        C = f * C + i * gg
        return o * jnp.tanh(C), C

    H = jnp.zeros((T, SE), _F32)
    C = jnp.zeros((T, SE), _F32)
    gens = [gcn_gen(t) for t in range(T)]
    starts = (16, 44, 72, 100)
    late_gins = [(t, b) for b in range(1, D // GB) for t in range(T)]
    for s in range(D):
        gin = jnp.concatenate([g[s:s + 1, :] for g in gin_scr], axis=0)
        H, C = substep(gin, H, C)
        if late_gins:
            gin_block(*late_gins.pop(0))
        for t in range(T):
            if s >= starts[t]:
                next(gens[t], None)
    for g in gens:
        for _ in g:
            pass
    H1 = H
    gvs = [gvs[t] for t in range(T)]

    sg = jnp.concatenate([H1, jnp.concatenate(gvs, axis=0)], axis=1)
    sv = jax.nn.relu(_mm(sg, Wsw_ref[...], ((1,), (0,))) + bsw_ref[...])

    W_hh2 = W_hh2_ref[...]
    gin2 = _mm(sv, W_ih2_ref[...], ((1,), (1,))) + bg2_ref[...]
    h = jnp.zeros((1, SE), _F32)
    c = jnp.zeros((1, SE), _F32)
    for s in range(T):
        g = gin2[s:s + 1] + _mm(h, W_hh2, ((1,), (1,)))
        i = _sig(g[:, :SE])
        f = _sig(g[:, SE:2 * SE])
        gg = jnp.tanh(g[:, 2 * SE:3 * SE])
        o = _sig(g[:, 3 * SE:])
        c = f * c + i * gg
        h = o * jnp.tanh(c)

    logits = _mm(h, Wc_ref[...], ((1,), (0,))) + bc_ref[...]
    m = jnp.max(logits, axis=1, keepdims=True)
    z = logits - m
    out_ref[...] = z - jnp.log(jnp.sum(jnp.exp(z), axis=1, keepdims=True))


def _forward(adj_mat_array, node_att_array, W1, b1, W2, b2, W_ih1, W_hh1,
             b_ih1, b_hh1, Wsw, bsw, W_ih2, W_hh2, b_ih2, b_hh2, Wc, bc,
             interpret=False):
    bg1 = (b_ih1 + b_hh1).reshape(1, -1)
    bg2 = (b_ih2 + b_hh2).reshape(1, -1)
    vmem = pl.BlockSpec(memory_space=pltpu.MemorySpace.VMEM)
    return pl.pallas_call(
        _body,
        out_shape=jax.ShapeDtypeStruct((1, NC), _F32),
        in_specs=[vmem] * 15
        + [pl.BlockSpec(memory_space=pltpu.MemorySpace.HBM)],
        scratch_shapes=[pltpu.VMEM((D, 4 * SE), _F32) for _ in range(T)]
        + [pltpu.VMEM((T, N, N), _F32), pltpu.SemaphoreType.DMA((T,))],
        interpret=interpret,
    )(node_att_array, W_ih1, bg1, W_hh1, W1, b1.reshape(1, -1), W2,
      b2.reshape(1, -1), Wsw, bsw.reshape(1, -1), W_ih2, W_hh2, bg2,
      Wc, bc.reshape(1, -1), adj_mat_array)


def kernel(adj_mat_array, node_att_array, W1, b1, W2, b2, W_ih1, W_hh1,
           b_ih1, b_hh1, Wsw, bsw, W_ih2, W_hh2, b_ih2, b_hh2, Wc, bc):
    return _forward(adj_mat_array, node_att_array, W1, b1, W2, b2, W_ih1,
                    W_hh1, b_ih1, b_hh1, Wsw, bsw, W_ih2, W_hh2, b_ih2,
                    b_hh2, Wc, bc)


# speedup vs baseline: 1.0788x; 1.0164x over previous
import functools

import jax
import jax.numpy as jnp
from jax.experimental import pallas as pl
from jax.experimental.pallas import tpu as pltpu

N = 512
D = 128
T = 4
GH = 64
NE = 64
SE = 128
NC = 10

_F32 = jnp.float32


def _mm(a, b, dims):
    return jax.lax.dot_general(
        a, b, dimension_numbers=(dims, ((), ())),
        preferred_element_type=_F32)


def _mmb(a, b, dims):
    return jax.lax.dot_general(
        a.astype(jnp.bfloat16), b.astype(jnp.bfloat16),
        dimension_numbers=(dims, ((), ())),
        preferred_element_type=_F32)


def _sig(x):
    return 0.5 * jnp.tanh(0.5 * x) + 0.5


def _body(x_ref, W_ih1_ref, bg1_ref, W_hh1_ref, W1_ref, b1_ref, W2_ref,
          b2_ref, Wsw_ref, bsw_ref, W_ih2_ref, W_hh2_ref, bg2_ref,
          Wc_ref, bc_ref, adj_hbm_ref, out_ref, gin0, gin1, gin2, gin3,
          adj_ref, adj_sem):
    gin_scr = (gin0, gin1, gin2, gin3)
    adj_copies = [
        pltpu.make_async_copy(adj_hbm_ref.at[t], adj_ref.at[t],
                              adj_sem.at[t])
        for t in range(T)]
    for cp in adj_copies:
        cp.start()
    W1 = W1_ref[...]
    W2 = W2_ref[...]
    W_hh1 = W_hh1_ref[...]

    CH = N // 4
    rl = jax.lax.broadcasted_iota(jnp.int32, (CH, N), 0)
    cl = jax.lax.broadcasted_iota(jnp.int32, (CH, N), 1)
    eye_c = [rl + c * CH == cl for c in range(4)]
    ones_col = jnp.ones((CH, 1), _F32)

    cidx = jax.lax.broadcasted_iota(jnp.int32, (1, 4 * SE), 1)
    gscale = jnp.where(
        jnp.logical_and(cidx >= 2 * SE, cidx < 3 * SE), 1.0, 0.5)
    for t in range(T):
        gin_scr[t][...] = (_mmb(x_ref[t], W_ih1_ref[...], ((0,), (1,)))
                           + bg1_ref[...]) * gscale

    gvs = {}

    def gcn_gen(t):
        adj_copies[t].wait()
        yield
        Awbs = []
        deg = jnp.zeros((N, 1), _F32)
        for c in range(4):
            a = adj_ref[t, c * CH:(c + 1) * CH, :]
            Aw_c = jnp.where(a > 0, a, 0.0)
            diag = jnp.sum(jnp.where(eye_c[c], Aw_c, 0.0), axis=1,
                           keepdims=True)
            Aw_c = Aw_c + jnp.where(
                eye_c[c], jnp.where(diag > 0, 0.0, 1.0), 0.0)
            deg = deg + _mm(Aw_c, ones_col, ((0,), (0,)))
            Awbs.append(Aw_c.astype(jnp.bfloat16))
            yield
        dinv = jnp.where(deg > 0, jax.lax.rsqrt(deg), 0.0)
        yield
        h = _mmb(x_ref[t], W1, ((1,), (0,)))
        yield
        for W_next, b in ((W2, b1_ref), (None, b2_ref)):
            z = dinv * h
            acc = _mmb(Awbs[0], z[:CH], ((0,), (0,)))
            yield
            for c in range(1, 4):
                acc = acc + _mmb(Awbs[c], z[c * CH:(c + 1) * CH],
                                 ((0,), (0,)))
                yield
            h = jax.nn.relu(dinv * acc + b[...])
            yield
            if W_next is not None:
                h = _mmb(h, W_next, ((1,), (0,)))
                yield
        gvs[t] = jnp.mean(h, axis=0, keepdims=True)
        yield

    gidx = jax.lax.broadcasted_iota(jnp.int32, (4 * SE, 1), 0)
    in_gg = jnp.logical_and(gidx >= 2 * SE, gidx < 3 * SE)
    W_hh1b = (W_hh1 * jnp.where(in_gg, 1.0, 0.5)).astype(jnp.bfloat16)

    def substep(gin, H, C):
        g = gin + _mmb(H, W_hh1b, ((1,), (1,)))
        th = jnp.tanh(g)
        i = 0.5 * th[:, :SE] + 0.5
        f = 0.5 * th[:, SE:2 * SE] + 0.5
        gg = th[:, 2 * SE:3 * SE]
        o = 0.5 * th[:, 3 * SE:] + 0.5
        C = f * C + i * gg
        return o * jnp.tanh(C), C

    H = jnp.zeros((T, SE), _F32)
    C = jnp.zeros((T, SE), _F32)
    gens = [gcn_gen(t) for t in range(T)]
    starts = (16, 44, 72, 100)
    for s in range(D):
        gin = jnp.concatenate([g[s:s + 1, :] for g in gin_scr], axis=0)
        H, C = substep(gin, H, C)
        for t in range(T):
            if s >= starts[t]:
                next(gens[t], None)
    for g in gens:
        for _ in g:
            pass
    H1 = H
    gvs = [gvs[t] for t in range(T)]

    sg = jnp.concatenate([H1, jnp.concatenate(gvs, axis=0)], axis=1)
    sv = jax.nn.relu(_mm(sg, Wsw_ref[...], ((1,), (0,))) + bsw_ref[...])

    W_hh2 = W_hh2_ref[...]
    gin2 = _mm(sv, W_ih2_ref[...], ((1,), (1,))) + bg2_ref[...]
    h = jnp.zeros((1, SE), _F32)
    c = jnp.zeros((1, SE), _F32)
    for s in range(T):
        g = gin2[s:s + 1] + _mm(h, W_hh2, ((1,), (1,)))
        i = _sig(g[:, :SE])
        f = _sig(g[:, SE:2 * SE])
        gg = jnp.tanh(g[:, 2 * SE:3 * SE])
        o = _sig(g[:, 3 * SE:])
        c = f * c + i * gg
        h = o * jnp.tanh(c)

    logits = _mm(h, Wc_ref[...], ((1,), (0,))) + bc_ref[...]
    m = jnp.max(logits, axis=1, keepdims=True)
    z = logits - m
    out_ref[...] = z - jnp.log(jnp.sum(jnp.exp(z), axis=1, keepdims=True))


def _forward(adj_mat_array, node_att_array, W1, b1, W2, b2, W_ih1, W_hh1,
             b_ih1, b_hh1, Wsw, bsw, W_ih2, W_hh2, b_ih2, b_hh2, Wc, bc,
             interpret=False):
    bg1 = (b_ih1 + b_hh1).reshape(1, -1)
    bg2 = (b_ih2 + b_hh2).reshape(1, -1)
    vmem = pl.BlockSpec(memory_space=pltpu.MemorySpace.VMEM)
    return pl.pallas_call(
        _body,
        out_shape=jax.ShapeDtypeStruct((1, NC), _F32),
        in_specs=[vmem] * 15
        + [pl.BlockSpec(memory_space=pltpu.MemorySpace.HBM)],
        scratch_shapes=[pltpu.VMEM((D, 4 * SE), _F32) for _ in range(T)]
        + [pltpu.VMEM((T, N, N), _F32), pltpu.SemaphoreType.DMA((T,))],
        interpret=interpret,
    )(node_att_array, W_ih1, bg1, W_hh1, W1, b1.reshape(1, -1), W2,
      b2.reshape(1, -1), Wsw, bsw.reshape(1, -1), W_ih2, W_hh2, bg2,
      Wc, bc.reshape(1, -1), adj_mat_array)


def kernel(adj_mat_array, node_att_array, W1, b1, W2, b2, W_ih1, W_hh1,
           b_ih1, b_hh1, Wsw, bsw, W_ih2, W_hh2, b_ih2, b_hh2, Wc, bc):
    return _forward(adj_mat_array, node_att_array, W1, b1, W2, b2, W_ih1,
                    W_hh1, b_ih1, b_hh1, Wsw, bsw, W_ih2, W_hh2, b_ih2,
                    b_hh2, Wc, bc)


# speedup vs baseline: 1.0860x; 1.0066x over previous
import jax
import jax.numpy as jnp
from jax.experimental import pallas as pl
from jax.experimental.pallas import tpu as pltpu

N = 512
D = 128
T = 4
GH = 64
NE = 64
SE = 128
NC = 10

_F32 = jnp.float32


def _mm(a, b, dims):
    return jax.lax.dot_general(
        a, b, dimension_numbers=(dims, ((), ())),
        preferred_element_type=_F32)


def _mmb(a, b, dims):
    return jax.lax.dot_general(
        a.astype(jnp.bfloat16), b.astype(jnp.bfloat16),
        dimension_numbers=(dims, ((), ())),
        preferred_element_type=_F32)


def _sig(x):
    return 0.5 * jnp.tanh(0.5 * x) + 0.5


def _body(x_ref, W_ih1_ref, bg1_ref, W_hh1_ref, W1_ref, b1_ref, W2_ref,
          b2_ref, Wsw_ref, bsw_ref, W_ih2_ref, W_hh2_ref, bg2_ref,
          Wc_ref, bc_ref, adj_hbm_ref, out_ref, gin0, gin1, gin2, gin3,
          adj_ref, adj_sem):
    gin_scr = (gin0, gin1, gin2, gin3)
    adj_copies = [
        pltpu.make_async_copy(adj_hbm_ref.at[t], adj_ref.at[t],
                              adj_sem.at[t])
        for t in range(T)]
    for cp in adj_copies:
        cp.start()
    W1 = W1_ref[...]
    W2 = W2_ref[...]
    W_hh1 = W_hh1_ref[...]

    CH = N // 4
    rl = jax.lax.broadcasted_iota(jnp.int32, (CH, N), 0)
    cl = jax.lax.broadcasted_iota(jnp.int32, (CH, N), 1)
    eye_c = [rl + c * CH == cl for c in range(4)]
    ones_col = jnp.ones((CH, 1), _F32)

    cidx = jax.lax.broadcasted_iota(jnp.int32, (1, 4 * SE), 1)
    gscale = jnp.where(
        jnp.logical_and(cidx >= 2 * SE, cidx < 3 * SE), 1.0, 0.5)
    for t in range(T):
        gin_scr[t][...] = (_mmb(x_ref[t], W_ih1_ref[...], ((0,), (1,)))
                           + bg1_ref[...]) * gscale

    gvs = {}

    def gcn_gen(t):
        adj_copies[t].wait()
        yield
        Awbs = []
        deg = jnp.zeros((N, 1), _F32)
        for c in range(4):
            a = adj_ref[t, c * CH:(c + 1) * CH, :]
            Aw_c = jnp.where(a > 0, a, 0.0)
            diag = jnp.sum(jnp.where(eye_c[c], Aw_c, 0.0), axis=1,
                           keepdims=True)
            Aw_c = Aw_c + jnp.where(
                eye_c[c], jnp.where(diag > 0, 0.0, 1.0), 0.0)
            deg = deg + _mm(Aw_c, ones_col, ((0,), (0,)))
            Awbs.append(Aw_c.astype(jnp.bfloat16))
            yield
        dinv = jnp.where(deg > 0, jax.lax.rsqrt(deg), 0.0)
        yield
        h = _mmb(x_ref[t], W1, ((1,), (0,)))
        yield
        for W_next, b in ((W2, b1_ref), (None, b2_ref)):
            z = dinv * h
            acc = _mmb(Awbs[0], z[:CH], ((0,), (0,)))
            yield
            for c in range(1, 4):
                acc = acc + _mmb(Awbs[c], z[c * CH:(c + 1) * CH],
                                 ((0,), (0,)))
                yield
            h = jax.nn.relu(dinv * acc + b[...])
            yield
            if W_next is not None:
                h = _mmb(h, W_next, ((1,), (0,)))
                yield
        gvs[t] = jnp.mean(h, axis=0, keepdims=True)
        yield

    gidx = jax.lax.broadcasted_iota(jnp.int32, (4 * SE, 1), 0)
    in_gg = jnp.logical_and(gidx >= 2 * SE, gidx < 3 * SE)
    W_hh1b = (W_hh1 * jnp.where(in_gg, 1.0, 0.5)).astype(jnp.bfloat16)

    def substep(gin, H, C):
        g = gin + _mmb(H, W_hh1b, ((1,), (1,)))
        th = jnp.tanh(g)
        i = 0.5 * th[:, :SE] + 0.5
        f = 0.5 * th[:, SE:2 * SE] + 0.5
        gg = th[:, 2 * SE:3 * SE]
        o = 0.5 * th[:, 3 * SE:] + 0.5
        C = f * C + i * gg
        return o * jnp.tanh(C), C

    H = jnp.zeros((T, SE), _F32)
    C = jnp.zeros((T, SE), _F32)
    gens = [gcn_gen(t) for t in range(T)]
    starts = (16, 44, 72, 100)
    for s in range(D):
        gin = jnp.concatenate([g[s:s + 1, :] for g in gin_scr], axis=0)
        H, C = substep(gin, H, C)
        for t in range(T):
            if s >= starts[t]:
                next(gens[t], None)
    for g in gens:
        for _ in g:
            pass
    H1 = H
    gvs = [gvs[t] for t in range(T)]

    sg = jnp.concatenate([H1, jnp.concatenate(gvs, axis=0)], axis=1)
    sv = jax.nn.relu(_mm(sg, Wsw_ref[...], ((1,), (0,))) + bsw_ref[...])

    W_hh2 = W_hh2_ref[...]
    gin2 = _mm(sv, W_ih2_ref[...], ((1,), (1,))) + bg2_ref[...]
    h = jnp.zeros((1, SE), _F32)
    c = jnp.zeros((1, SE), _F32)
    for s in range(T):
        g = gin2[s:s + 1] + _mm(h, W_hh2, ((1,), (1,)))
        i = _sig(g[:, :SE])
        f = _sig(g[:, SE:2 * SE])
        gg = jnp.tanh(g[:, 2 * SE:3 * SE])
        o = _sig(g[:, 3 * SE:])
        c = f * c + i * gg
        h = o * jnp.tanh(c)

    logits = _mm(h, Wc_ref[...], ((1,), (0,))) + bc_ref[...]
    m = jnp.max(logits, axis=1, keepdims=True)
    z = logits - m
    out_ref[...] = z - jnp.log(jnp.sum(jnp.exp(z), axis=1, keepdims=True))


def _forward(adj_mat_array, node_att_array, W1, b1, W2, b2, W_ih1, W_hh1,
             b_ih1, b_hh1, Wsw, bsw, W_ih2, W_hh2, b_ih2, b_hh2, Wc, bc,
             interpret=False):
    bg1 = (b_ih1 + b_hh1).reshape(1, -1)
    bg2 = (b_ih2 + b_hh2).reshape(1, -1)
    vmem = pl.BlockSpec(memory_space=pltpu.MemorySpace.VMEM)
    return pl.pallas_call(
        _body,
        out_shape=jax.ShapeDtypeStruct((1, NC), _F32),
        in_specs=[vmem] * 15
        + [pl.BlockSpec(memory_space=pltpu.MemorySpace.HBM)],
        scratch_shapes=[pltpu.VMEM((D, 4 * SE), _F32) for _ in range(T)]
        + [pltpu.VMEM((T, N, N), _F32), pltpu.SemaphoreType.DMA((T,))],
        interpret=interpret,
    )(node_att_array, W_ih1, bg1, W_hh1, W1, b1.reshape(1, -1), W2,
      b2.reshape(1, -1), Wsw, bsw.reshape(1, -1), W_ih2, W_hh2, bg2,
      Wc, bc.reshape(1, -1), adj_mat_array)


def kernel(adj_mat_array, node_att_array, W1, b1, W2, b2, W_ih1, W_hh1,
           b_ih1, b_hh1, Wsw, bsw, W_ih2, W_hh2, b_ih2, b_hh2, Wc, bc):
    return _forward(adj_mat_array, node_att_array, W1, b1, W2, b2, W_ih1,
                    W_hh1, b_ih1, b_hh1, Wsw, bsw, W_ih2, W_hh2, b_ih2,
                    b_hh2, Wc, bc)
